# baseline (speedup 1.0000x reference)
import jax
import jax.numpy as jnp
from jax.experimental import pallas as pl
from jax.experimental.pallas import tpu as pltpu

LANES = 128


def _make_body(T, S, A):
    f32 = jnp.float32

    def body(*refs):
        sr_ref = refs[0]
        hds_ref, acts_ref, vals_ref = refs[1:4]
        w1s_ref, wh_ref, b1_ref, w2_ref, b2_ref = refs[4:9]
        out_ref, val_ref = refs[9:]

        w1s = w1s_ref[...]
        b1 = b1_ref[...]
        w2 = w2_ref[...]
        b2 = b2_ref[...]
        wh = [wh_ref[i:i + 1] for i in range(3)]

        outs = []
        for t in range(T):
            x = sr_ref[:, t, :]
            hp = jnp.dot(x, w1s, preferred_element_type=f32)
            hp = (hp
                  + hds_ref[:, t:t + 1] * wh[0]
                  + acts_ref[:, t:t + 1] * wh[1]
                  + vals_ref[:, t:t + 1] * wh[2]
                  + b1)
            h = jnp.tanh(hp)
            outs.append(jnp.dot(h, w2, preferred_element_type=f32) + b2)

        val_ref[...] = outs[0][:, A:A + 1]

        m = outs[0]
        for t in range(1, T):
            m = jnp.maximum(m, outs[t])
        se = jnp.exp(outs[0] - m)
        for t in range(1, T):
            se = se + jnp.exp(outs[t] - m)
        lse = m + jnp.log(se)
        for t in range(T):
            out_ref[:, t, :] = (outs[t] - lse)[:, :A]

    return body


def _pick_b_block(B):
    for cand in (512, 256, 128, 64, 32, 16, 8):
        if B % cand == 0 and (B // cand) >= 2:
            return cand
    return B


def kernel(w1a, b1a, w2a, b2a, w1c, b1c, w2c, b2c, SR, HDs, acts, values):
    f32 = jnp.float32
    B, T, S = SR.shape
    H = w1a.shape[1]
    H2 = 2 * H
    A = w2a.shape[1]

    w1f = jnp.concatenate([w1a, w1c], axis=1).astype(f32)
    w1s = w1f[:S]
    wh = w1f[S:S + 3]
    b1f = jnp.concatenate([b1a, b1c], axis=1).astype(f32)
    w2f = jnp.zeros((H2, LANES), f32)
    w2f = w2f.at[:H, :A].set(w2a.astype(f32))
    w2f = w2f.at[H:, A:A + 1].set(w2c.astype(f32))
    b2f = jnp.zeros((1, LANES), f32)
    b2f = b2f.at[:, :A].set(b2a.astype(f32))
    b2f = b2f.at[:, A:A + 1].set(b2c.astype(f32))

    SRf = SR.astype(f32)
    HDsf = HDs.astype(f32)
    actsf = acts.astype(f32)
    valsf = values.astype(f32)

    b_block = _pick_b_block(B)
    nb = B // b_block
    row_tile = lambda i: (i, 0)
    full = lambda i: (0, 0)

    logp, val = pl.pallas_call(
        _make_body(T, S, A),
        grid=(nb,),
        in_specs=(
            [pl.BlockSpec((b_block, T, S), lambda i: (i, 0, 0))]
            + [pl.BlockSpec((b_block, T), row_tile)] * 3
            + [
                pl.BlockSpec((S, H2), full),
                pl.BlockSpec((3, H2), full),
                pl.BlockSpec((1, H2), full),
                pl.BlockSpec((H2, LANES), full),
                pl.BlockSpec((1, LANES), full),
            ]
        ),
        out_specs=[
            pl.BlockSpec((b_block, T, A), lambda i: (i, 0, 0)),
            pl.BlockSpec((b_block, 1), row_tile),
        ],
        out_shape=[
            jax.ShapeDtypeStruct((B, T, A), f32),
            jax.ShapeDtypeStruct((B, 1), f32),
        ],
        compiler_params=pltpu.CompilerParams(
            dimension_semantics=("parallel",)),
    )(SRf, HDsf, actsf, valsf, w1s, wh, b1f, w2f, b2f)

    value = val.reshape(B)
    return logp, value


# speedup vs baseline: 3.5503x; 3.5503x over previous
import jax
import jax.numpy as jnp
from jax.experimental import pallas as pl
from jax.experimental.pallas import tpu as pltpu


def _make_body(T, S, A, H2):
    f32 = jnp.float32

    def body(sr_ref, hds_ref, acts_ref, vals_ref,
             w4_ref, wf_ref, b1_ref, w2_ref, b2_ref,
             out_ref, val_ref):
        x2d = sr_ref[...].reshape(S * T, sr_ref.shape[-1])
        hp4 = jnp.dot(w4_ref[...], x2d, preferred_element_type=f32)

        feats = jnp.concatenate(
            [hds_ref[...], acts_ref[...], vals_ref[...]], axis=0)
        hp4 = hp4 + jnp.dot(wf_ref[...], feats, preferred_element_type=f32)

        b1 = b1_ref[...]
        w2 = w2_ref[...]
        b2 = b2_ref[...]

        outs = []
        for t in range(T):
            h = jnp.tanh(hp4[H2 * t:H2 * (t + 1), :] + b1)
            outs.append(jnp.dot(w2, h, preferred_element_type=f32) + b2)

        val_ref[...] = outs[0][A:A + 1, :]

        m = outs[0]
        for t in range(1, T):
            m = jnp.maximum(m, outs[t])
        se = jnp.exp(outs[0] - m)
        for t in range(1, T):
            se = se + jnp.exp(outs[t] - m)
        lse = m + jnp.log(se)
        for t in range(T):
            out_ref[t, :, :] = (outs[t] - lse)[:A, :]

    return body


def _pick_b_block(B):
    for cand in (512, 256, 128):
        if B % cand == 0 and (B // cand) >= 2:
            return cand
    return B


def kernel(w1a, b1a, w2a, b2a, w1c, b1c, w2c, b2c, SR, HDs, acts, values):
    f32 = jnp.float32
    B, T, S = SR.shape
    H = w1a.shape[1]
    H2 = 2 * H
    A = w2a.shape[1]

    w1f = jnp.concatenate([w1a, w1c], axis=1).astype(f32)
    w4 = jnp.zeros((T, H2, S, T), f32)
    w4 = w4.at[jnp.arange(T), :, :, jnp.arange(T)].set(
        jnp.broadcast_to(w1f[:S].T, (T, H2, S)))
    w4 = w4.reshape(T * H2, S * T)
    wf = jnp.zeros((T, H2, 3, T), f32)
    wf = wf.at[jnp.arange(T), :, :, jnp.arange(T)].set(
        jnp.broadcast_to(w1f[S:S + 3].T, (T, H2, 3)))
    wf = wf.reshape(T * H2, 3 * T)

    b1f = jnp.concatenate([b1a, b1c], axis=1).astype(f32).T
    w2f = jnp.zeros((H2, H2), f32)
    w2f = w2f.at[:A, :H].set(w2a.astype(f32).T)
    w2f = w2f.at[A:A + 1, H:].set(w2c.astype(f32).T)
    b2f = jnp.zeros((H2, 1), f32)
    b2f = b2f.at[:A, 0].set(b2a.astype(f32)[0])
    b2f = b2f.at[A, 0].set(b2c.astype(f32)[0, 0])

    SRt = jnp.transpose(SR.astype(f32), (2, 1, 0))
    HDst = HDs.astype(f32).T
    actst = acts.astype(f32).T
    valst = values.astype(f32).T

    bb = _pick_b_block(B)
    nb = B // bb
    lane_tile2 = lambda i: (0, i)
    full2 = lambda i: (0, 0)

    out_tab, val = pl.pallas_call(
        _make_body(T, S, A, H2),
        grid=(nb,),
        in_specs=[
            pl.BlockSpec((S, T, bb), lambda i: (0, 0, i)),
            pl.BlockSpec((T, bb), lane_tile2),
            pl.BlockSpec((T, bb), lane_tile2),
            pl.BlockSpec((T, bb), lane_tile2),
            pl.BlockSpec((T * H2, S * T), full2),
            pl.BlockSpec((T * H2, 3 * T), full2),
            pl.BlockSpec((H2, 1), full2),
            pl.BlockSpec((H2, H2), full2),
            pl.BlockSpec((H2, 1), full2),
        ],
        out_specs=[
            pl.BlockSpec((T, A, bb), lambda i: (0, 0, i)),
            pl.BlockSpec((1, bb), lane_tile2),
        ],
        out_shape=[
            jax.ShapeDtypeStruct((T, A, B), f32),
            jax.ShapeDtypeStruct((1, B), f32),
        ],
        compiler_params=pltpu.CompilerParams(
            dimension_semantics=("parallel",)),
    )(SRt, HDst, actst, valst, w4, wf, b1f, w2f, b2f)

    logp = jnp.transpose(out_tab, (2, 0, 1))
    value = val.reshape(B)
    return logp, value


# speedup vs baseline: 4.7511x; 1.3382x over previous
import jax
import jax.numpy as jnp
from jax.experimental import pallas as pl
from jax.experimental.pallas import tpu as pltpu


def _make_body(T, S, A, H, H2):
    f32 = jnp.float32

    def body(sr_ref, hds_ref, acts_ref, vals_ref,
             w4_ref, wf_ref, b1_ref, w2a_ref, b2a_ref, w2c_ref, b2c_ref,
             out_ref, val_ref):
        x2d = sr_ref[...].reshape(S * T, sr_ref.shape[-1])
        hp4 = jnp.dot(w4_ref[...], x2d, preferred_element_type=f32)

        feats = jnp.concatenate(
            [hds_ref[...], acts_ref[...], vals_ref[...]], axis=0)
        hp4 = hp4 + jnp.dot(wf_ref[...], feats, preferred_element_type=f32)

        b1 = b1_ref[...]
        w2a = w2a_ref[...]
        b2a = b2a_ref[...]

        outs = []
        for t in range(T):
            ha = jnp.tanh(hp4[H2 * t:H2 * t + H, :] + b1[:H])
            outs.append(jnp.dot(w2a, ha, preferred_element_type=f32) + b2a)

        hc = jnp.tanh(hp4[H:H2, :] + b1[H:])
        val_ref[...] = (jnp.dot(w2c_ref[...], hc, preferred_element_type=f32)
                        + b2c_ref[...])[0:1, :]

        m = outs[0]
        for t in range(1, T):
            m = jnp.maximum(m, outs[t])
        se = jnp.exp(outs[0] - m)
        for t in range(1, T):
            se = se + jnp.exp(outs[t] - m)
        lse = m + jnp.log(se)
        for t in range(T):
            out_ref[t, :, :] = outs[t] - lse

    return body


def _pick_b_block(B):
    for cand in (1024, 512, 256, 128):
        if B % cand == 0 and (B // cand) >= 2:
            return cand
    return B


def kernel(w1a, b1a, w2a, b2a, w1c, b1c, w2c, b2c, SR, HDs, acts, values):
    f32 = jnp.float32
    B, T, S = SR.shape
    H = w1a.shape[1]
    H2 = 2 * H
    A = w2a.shape[1]

    w1f = jnp.concatenate([w1a, w1c], axis=1).astype(f32)
    eyeT = jnp.eye(T, dtype=f32)
    w4 = (w1f[:S].T[None, :, :, None] * eyeT[:, None, None, :]
          ).reshape(T * H2, S * T)
    wf = (w1f[S:S + 3].T[None, :, :, None] * eyeT[:, None, None, :]
          ).reshape(T * H2, 3 * T)

    b1f = jnp.concatenate([b1a, b1c], axis=1).astype(f32).T
    w2at = w2a.astype(f32).T
    b2at = b2a.astype(f32).T
    w2ct = w2c.astype(f32).T
    b2ct = b2c.astype(f32)

    SRt = jnp.transpose(SR.astype(f32), (2, 1, 0))
    HDst = HDs.astype(f32).T
    actst = acts.astype(f32).T
    valst = values.astype(f32).T

    bb = _pick_b_block(B)
    nb = B // bb
    lane_tile2 = lambda i: (0, i)
    full2 = lambda i: (0, 0)

    out_tab, val = pl.pallas_call(
        _make_body(T, S, A, H, H2),
        grid=(nb,),
        in_specs=[
            pl.BlockSpec((S, T, bb), lambda i: (0, 0, i)),
            pl.BlockSpec((T, bb), lane_tile2),
            pl.BlockSpec((T, bb), lane_tile2),
            pl.BlockSpec((T, bb), lane_tile2),
            pl.BlockSpec((T * H2, S * T), full2),
            pl.BlockSpec((T * H2, 3 * T), full2),
            pl.BlockSpec((H2, 1), full2),
            pl.BlockSpec((A, H), full2),
            pl.BlockSpec((A, 1), full2),
            pl.BlockSpec((1, H), full2),
            pl.BlockSpec((1, 1), full2),
        ],
        out_specs=[
            pl.BlockSpec((T, A, bb), lambda i: (0, 0, i)),
            pl.BlockSpec((1, bb), lane_tile2),
        ],
        out_shape=[
            jax.ShapeDtypeStruct((T, A, B), f32),
            jax.ShapeDtypeStruct((1, B), f32),
        ],
        compiler_params=pltpu.CompilerParams(
            dimension_semantics=("parallel",)),
    )(SRt, HDst, actst, valst, w4, wf, b1f, w2at, b2at, w2ct, b2ct)

    logp = jnp.transpose(out_tab, (2, 0, 1))
    value = val.reshape(B)
    return logp, value


# speedup vs baseline: 4.9246x; 1.0365x over previous
import jax
import jax.numpy as jnp
from jax.experimental import pallas as pl
from jax.experimental.pallas import tpu as pltpu

_LHS_T = (((0,), (0,)), ((), ()))


def _make_body(T, S, A, H, H2):
    f32 = jnp.float32

    def body(sr_ref, hds_ref, acts_ref, vals_ref,
             w4_ref, wf_ref, w2a_ref, w2c_ref, b2c_ref,
             out_ref, val_ref):
        bb = sr_ref.shape[-1]
        x2d = sr_ref[...].reshape(S * T, bb)
        hp4 = jnp.dot(w4_ref[...], x2d, preferred_element_type=f32)

        feats = jnp.concatenate(
            [hds_ref[...], acts_ref[...], vals_ref[...],
             jnp.ones((T, bb), f32)], axis=0)
        hp4 = hp4 + jnp.dot(wf_ref[...], feats, preferred_element_type=f32)

        w2a = w2a_ref[...]

        outs = []
        for t in range(T):
            ha = jnp.tanh(hp4[H2 * t:H2 * t + H, :])
            outs.append(jax.lax.dot_general(
                w2a, ha, _LHS_T, preferred_element_type=f32))

        hc = jnp.tanh(hp4[H:H2, :])
        val_ref[...] = (jax.lax.dot_general(
            w2c_ref[...], hc, _LHS_T, preferred_element_type=f32)
            + b2c_ref[...])

        m = outs[0]
        for t in range(1, T):
            m = jnp.maximum(m, outs[t])
        se = jnp.exp(outs[0] - m)
        for t in range(1, T):
            se = se + jnp.exp(outs[t] - m)
        lse = m + jnp.log(se)
        for t in range(T):
            out_ref[t, :, :] = outs[t] - lse

    return body


def _pick_b_block(B):
    for cand in (1024, 512, 256, 128):
        if B % cand == 0 and (B // cand) >= 2:
            return cand
    return B


def kernel(w1a, b1a, w2a, b2a, w1c, b1c, w2c, b2c, SR, HDs, acts, values):
    f32 = jnp.float32
    B, T, S = SR.shape
    H = w1a.shape[1]
    H2 = 2 * H
    A = w2a.shape[1]

    w1f = jnp.concatenate([w1a, w1c], axis=1).astype(f32)
    b1f = jnp.concatenate([b1a, b1c], axis=1).astype(f32)
    wsmall = jnp.concatenate([w1f[S:S + 3], b1f], axis=0)
    eyeT = jnp.eye(T, dtype=f32)
    w4 = (w1f[:S].T[None, :, :, None] * eyeT[:, None, None, :]
          ).reshape(T * H2, S * T)
    wf = (wsmall.T[None, :, :, None] * eyeT[:, None, None, :]
          ).reshape(T * H2, 4 * T)

    SRt = jnp.transpose(SR.astype(f32), (2, 1, 0))
    HDst = HDs.astype(f32).T
    actst = acts.astype(f32).T
    valst = values.astype(f32).T

    bb = _pick_b_block(B)
    nb = B // bb
    lane_tile2 = lambda i: (0, i)
    full2 = lambda i: (0, 0)

    out_tab, val = pl.pallas_call(
        _make_body(T, S, A, H, H2),
        grid=(nb,),
        in_specs=[
            pl.BlockSpec((S, T, bb), lambda i: (0, 0, i)),
            pl.BlockSpec((T, bb), lane_tile2),
            pl.BlockSpec((T, bb), lane_tile2),
            pl.BlockSpec((T, bb), lane_tile2),
            pl.BlockSpec((T * H2, S * T), full2),
            pl.BlockSpec((T * H2, 4 * T), full2),
            pl.BlockSpec((H, A), full2),
            pl.BlockSpec((H, 1), full2),
            pl.BlockSpec((1, 1), full2),
        ],
        out_specs=[
            pl.BlockSpec((T, A, bb), lambda i: (0, 0, i)),
            pl.BlockSpec((1, bb), lane_tile2),
        ],
        out_shape=[
            jax.ShapeDtypeStruct((T, A, B), f32),
            jax.ShapeDtypeStruct((1, B), f32),
        ],
        compiler_params=pltpu.CompilerParams(
            dimension_semantics=("parallel",),
            allow_input_fusion=[False, False, False, False,
                                True, True, True, True, True],
        ),
    )(SRt, HDst, actst, valst, w4, wf, w2a.astype(f32), w2c.astype(f32),
      b2c.astype(f32))

    logp = jnp.transpose(out_tab, (2, 0, 1))
    value = val.reshape(B)
    return logp, value


# speedup vs baseline: 5.2228x; 1.0606x over previous
import jax
import jax.numpy as jnp
from jax.experimental import pallas as pl
from jax.experimental.pallas import tpu as pltpu


def _make_body(T, S, A, H, H2):
    f32 = jnp.float32

    def body(sr_ref, hds_ref, acts_ref, vals_ref,
             w1at_ref, w1ct_ref, b1col_ref, w2at_ref, w2ct_ref, b2c_ref,
             out_ref, val_ref, w4s_ref, wfs_ref):
        bb = sr_ref.shape[-1]

        @pl.when(pl.program_id(0) == 0)
        def _build_interleaved_weights():
            w1fT = jnp.concatenate(
                [w1at_ref[...], w1ct_ref[...]], axis=0)
            wmain = w1fT[:, :S]
            wsmall = jnp.concatenate(
                [w1fT[:, S:S + 3], b1col_ref[...]], axis=1)
            for t in range(T):
                ci = jax.lax.broadcasted_iota(jnp.int32, (S, S * T), 1)
                ri = jax.lax.broadcasted_iota(jnp.int32, (S, S * T), 0)
                sel = (ci == T * ri + t).astype(f32)
                w4s_ref[H2 * t:H2 * (t + 1), :] = jnp.dot(
                    wmain, sel, preferred_element_type=f32)
                cif = jax.lax.broadcasted_iota(jnp.int32, (4, 4 * T), 1)
                rif = jax.lax.broadcasted_iota(jnp.int32, (4, 4 * T), 0)
                self_f = (cif == T * rif + t).astype(f32)
                wfs_ref[H2 * t:H2 * (t + 1), :] = jnp.dot(
                    wsmall, self_f, preferred_element_type=f32)

        x2d = sr_ref[...].reshape(S * T, bb)
        hp4 = jnp.dot(w4s_ref[...], x2d, preferred_element_type=f32)

        feats = jnp.concatenate(
            [hds_ref[...], acts_ref[...], vals_ref[...],
             jnp.ones((T, bb), f32)], axis=0)
        hp4 = hp4 + jnp.dot(wfs_ref[...], feats, preferred_element_type=f32)

        w2at = w2at_ref[...]

        outs = []
        for t in range(T):
            ha = jnp.tanh(hp4[H2 * t:H2 * t + H, :])
            outs.append(jnp.dot(w2at, ha, preferred_element_type=f32))

        hc = jnp.tanh(hp4[H:H2, :])
        val_ref[...] = (jnp.dot(w2ct_ref[...], hc, preferred_element_type=f32)
                        + b2c_ref[...])

        m = outs[0]
        for t in range(1, T):
            m = jnp.maximum(m, outs[t])
        se = jnp.exp(outs[0] - m)
        for t in range(1, T):
            se = se + jnp.exp(outs[t] - m)
        lse = m + jnp.log(se)
        for t in range(T):
            out_ref[t, :, :] = outs[t] - lse

    return body


def _pick_b_block(B):
    for cand in (1024, 512, 256, 128):
        if B % cand == 0 and (B // cand) >= 2:
            return cand
    return B


def kernel(w1a, b1a, w2a, b2a, w1c, b1c, w2c, b2c, SR, HDs, acts, values):
    f32 = jnp.float32
    B, T, S = SR.shape
    H = w1a.shape[1]
    H2 = 2 * H
    A = w2a.shape[1]

    b1col = jnp.concatenate([b1a, b1c], axis=1).astype(f32).T

    w1at = w1a.astype(f32).T
    w1ct = w1c.astype(f32).T
    w2at = w2a.astype(f32).T
    w2ct = w2c.astype(f32).T
    b2ct = b2c.astype(f32)

    SRt = jnp.transpose(SR.astype(f32), (2, 1, 0))
    HDst = HDs.astype(f32).T
    actst = acts.astype(f32).T
    valst = values.astype(f32).T

    bb = _pick_b_block(B)
    nb = B // bb
    lane_tile2 = lambda i: (0, i)
    full2 = lambda i: (0, 0)

    out_tab, val = pl.pallas_call(
        _make_body(T, S, A, H, H2),
        grid=(nb,),
        in_specs=[
            pl.BlockSpec((S, T, bb), lambda i: (0, 0, i)),
            pl.BlockSpec((T, bb), lane_tile2),
            pl.BlockSpec((T, bb), lane_tile2),
            pl.BlockSpec((T, bb), lane_tile2),
            pl.BlockSpec((H, S + 3), full2),
            pl.BlockSpec((H, S + 3), full2),
            pl.BlockSpec((H2, 1), full2),
            pl.BlockSpec((A, H), full2),
            pl.BlockSpec((1, H), full2),
            pl.BlockSpec((1, 1), full2),
        ],
        out_specs=[
            pl.BlockSpec((T, A, bb), lambda i: (0, 0, i)),
            pl.BlockSpec((1, bb), lane_tile2),
        ],
        out_shape=[
            jax.ShapeDtypeStruct((T, A, B), f32),
            jax.ShapeDtypeStruct((1, B), f32),
        ],
        scratch_shapes=[
            pltpu.VMEM((T * H2, S * T), f32),
            pltpu.VMEM((T * H2, 4 * T), f32),
        ],
        compiler_params=pltpu.CompilerParams(
            dimension_semantics=("arbitrary",)),
    )(SRt, HDst, actst, valst, w1at, w1ct, b1col, w2at, w2ct, b2ct)

    logp = jnp.transpose(out_tab, (2, 0, 1))
    value = val.reshape(B)
    return logp, value


# speedup vs baseline: 6.1059x; 1.1691x over previous
import jax
import jax.numpy as jnp
from jax.experimental import pallas as pl
from jax.experimental.pallas import tpu as pltpu


def _make_body(T, S, A, H, H2):
    f32 = jnp.float32

    def body(sr_ref, hds_ref, acts_ref, vals_ref,
             w1at_ref, w1ct_ref, b1col_ref, w2at_ref, w2ct_ref, b2c_ref,
             out_ref, val_ref, w4s_ref, wfs_ref):
        bb = sr_ref.shape[-1]

        @pl.when(pl.program_id(0) == 0)
        def _build_interleaved_weights():
            w1fT = jnp.concatenate(
                [w1at_ref[...], w1ct_ref[...]], axis=0)
            wmain = w1fT[:, :S]
            wsmall = jnp.concatenate(
                [w1fT[:, S:S + 3], b1col_ref[...]], axis=1)
            for t in range(T):
                ci = jax.lax.broadcasted_iota(jnp.int32, (S, S * T), 1)
                ri = jax.lax.broadcasted_iota(jnp.int32, (S, S * T), 0)
                sel = (ci == T * ri + t).astype(f32)
                w4s_ref[H2 * t:H2 * (t + 1), :] = jnp.dot(
                    wmain, sel, preferred_element_type=f32)
                cif = jax.lax.broadcasted_iota(jnp.int32, (4, 4 * T), 1)
                rif = jax.lax.broadcasted_iota(jnp.int32, (4, 4 * T), 0)
                self_f = (cif == T * rif + t).astype(f32)
                wfs_ref[H2 * t:H2 * (t + 1), :] = jnp.dot(
                    wsmall, self_f, preferred_element_type=f32)

        x2d = sr_ref[...].reshape(S * T, bb)
        hp4 = jnp.dot(w4s_ref[...], x2d, preferred_element_type=f32)

        feats = jnp.concatenate(
            [hds_ref[...], acts_ref[...], vals_ref[...],
             jnp.ones((T, bb), f32)], axis=0)
        hp4 = hp4 + jnp.dot(wfs_ref[...], feats, preferred_element_type=f32)

        w2at = w2at_ref[...]

        outs = []
        for t in range(T):
            ha = jnp.tanh(hp4[H2 * t:H2 * t + H, :])
            outs.append(jnp.dot(w2at, ha, preferred_element_type=f32))

        hc = jnp.tanh(hp4[H:H2, :])
        val_ref[...] = (jnp.dot(w2ct_ref[...], hc, preferred_element_type=f32)
                        + b2c_ref[...])

        m = outs[0]
        for t in range(1, T):
            m = jnp.maximum(m, outs[t])
        se = jnp.exp(outs[0] - m)
        for t in range(1, T):
            se = se + jnp.exp(outs[t] - m)
        lse = m + jnp.log(se)
        for t in range(T):
            out_ref[t, :, :] = outs[t] - lse

    return body


def _pick_b_block(B):
    for cand in (2048, 1024, 512, 256, 128):
        if B % cand == 0 and (B // cand) >= 2:
            return cand
    return B


def kernel(w1a, b1a, w2a, b2a, w1c, b1c, w2c, b2c, SR, HDs, acts, values):
    f32 = jnp.float32
    B, T, S = SR.shape
    H = w1a.shape[1]
    H2 = 2 * H
    A = w2a.shape[1]

    b1col = jnp.concatenate([b1a, b1c], axis=1).astype(f32).T

    w1at = w1a.astype(f32).T
    w1ct = w1c.astype(f32).T
    w2at = w2a.astype(f32).T
    w2ct = w2c.astype(f32).T
    b2ct = b2c.astype(f32)

    SRt = jnp.transpose(SR.astype(f32), (2, 1, 0))
    HDst = HDs.astype(f32).T
    actst = acts.astype(f32).T
    valst = values.astype(f32).T

    bb = _pick_b_block(B)
    nb = B // bb
    lane_tile2 = lambda i: (0, i)
    full2 = lambda i: (0, 0)

    out_tab, val = pl.pallas_call(
        _make_body(T, S, A, H, H2),
        grid=(nb,),
        in_specs=[
            pl.BlockSpec((S, T, bb), lambda i: (0, 0, i)),
            pl.BlockSpec((T, bb), lane_tile2),
            pl.BlockSpec((T, bb), lane_tile2),
            pl.BlockSpec((T, bb), lane_tile2),
            pl.BlockSpec((H, S + 3), full2),
            pl.BlockSpec((H, S + 3), full2),
            pl.BlockSpec((H2, 1), full2),
            pl.BlockSpec((A, H), full2),
            pl.BlockSpec((1, H), full2),
            pl.BlockSpec((1, 1), full2),
        ],
        out_specs=[
            pl.BlockSpec((T, A, bb), lambda i: (0, 0, i)),
            pl.BlockSpec((1, bb), lane_tile2),
        ],
        out_shape=[
            jax.ShapeDtypeStruct((T, A, B), f32),
            jax.ShapeDtypeStruct((1, B), f32),
        ],
        scratch_shapes=[
            pltpu.VMEM((T * H2, S * T), f32),
            pltpu.VMEM((T * H2, 4 * T), f32),
        ],
        compiler_params=pltpu.CompilerParams(
            dimension_semantics=("arbitrary",)),
    )(SRt, HDst, actst, valst, w1at, w1ct, b1col, w2at, w2ct, b2ct)

    logp = jnp.transpose(out_tab, (2, 0, 1))
    value = val.reshape(B)
    return logp, value


# speedup vs baseline: 6.1606x; 1.0090x over previous
import jax
import jax.numpy as jnp
from jax.experimental import pallas as pl
from jax.experimental.pallas import tpu as pltpu


def _make_body(T, S, A, H, H2):
    f32 = jnp.float32

    def body(sr_ref, hds_ref, acts_ref, vals_ref,
             w1at_ref, w1ct_ref, b1col_ref, w2at_ref, w2ct_ref, b2c_ref,
             out_ref, val_ref, w4s_ref, wfs_ref):
        bb = sr_ref.shape[-1]

        @pl.when(pl.program_id(0) == 0)
        def _build_interleaved_weights():
            w1fT = jnp.concatenate(
                [w1at_ref[...], w1ct_ref[...]], axis=0)
            wmain = w1fT[:, :S]
            wsmall = jnp.concatenate(
                [w1fT[:, S:S + 3], b1col_ref[...]], axis=1)
            for t in range(T):
                ci = jax.lax.broadcasted_iota(jnp.int32, (S, S * T), 1)
                ri = jax.lax.broadcasted_iota(jnp.int32, (S, S * T), 0)
                sel = (ci == T * ri + t).astype(f32)
                w4s_ref[H2 * t:H2 * (t + 1), :] = jnp.dot(
                    wmain, sel, preferred_element_type=f32)
                cif = jax.lax.broadcasted_iota(jnp.int32, (4, 4 * T), 1)
                rif = jax.lax.broadcasted_iota(jnp.int32, (4, 4 * T), 0)
                self_f = (cif == T * rif + t).astype(f32)
                wfs_ref[H2 * t:H2 * (t + 1), :] = jnp.dot(
                    wsmall, self_f, preferred_element_type=f32)

        x2d = sr_ref[...].reshape(S * T, bb)
        hp4 = jnp.dot(w4s_ref[...], x2d, preferred_element_type=f32)

        feats = jnp.concatenate(
            [hds_ref[...], acts_ref[...], vals_ref[...],
             jnp.ones((T, bb), f32)], axis=0)
        hp4 = hp4 + jnp.dot(wfs_ref[...], feats, preferred_element_type=f32)

        w2at = w2at_ref[...]

        outs = []
        for t in range(T):
            ha = jnp.tanh(hp4[H2 * t:H2 * t + H, :])
            outs.append(jnp.dot(w2at, ha, preferred_element_type=f32))

        hc = jnp.tanh(hp4[H:H2, :])
        val_ref[...] = (jnp.dot(w2ct_ref[...], hc, preferred_element_type=f32)
                        + b2c_ref[...])

        m = outs[0]
        for t in range(1, T):
            m = jnp.maximum(m, outs[t])
        se = jnp.exp(outs[0] - m)
        for t in range(1, T):
            se = se + jnp.exp(outs[t] - m)
        lse = m + jnp.log(se)
        for t in range(T):
            out_ref[t, :, :] = outs[t] - lse

    return body


def _pick_b_block(B):
    for cand in (4096, 2048, 1024, 512, 256, 128):
        if B % cand == 0 and (B // cand) >= 2:
            return cand
    return B


def kernel(w1a, b1a, w2a, b2a, w1c, b1c, w2c, b2c, SR, HDs, acts, values):
    f32 = jnp.float32
    B, T, S = SR.shape
    H = w1a.shape[1]
    H2 = 2 * H
    A = w2a.shape[1]

    b1col = jnp.concatenate([b1a, b1c], axis=1).astype(f32).T

    w1at = w1a.astype(f32).T
    w1ct = w1c.astype(f32).T
    w2at = w2a.astype(f32).T
    w2ct = w2c.astype(f32).T
    b2ct = b2c.astype(f32)

    SRt = jnp.transpose(SR.astype(f32), (2, 1, 0))
    HDst = HDs.astype(f32).T
    actst = acts.astype(f32).T
    valst = values.astype(f32).T

    bb = _pick_b_block(B)
    nb = B // bb
    lane_tile2 = lambda i: (0, i)
    full2 = lambda i: (0, 0)

    out_tab, val = pl.pallas_call(
        _make_body(T, S, A, H, H2),
        grid=(nb,),
        in_specs=[
            pl.BlockSpec((S, T, bb), lambda i: (0, 0, i)),
            pl.BlockSpec((T, bb), lane_tile2),
            pl.BlockSpec((T, bb), lane_tile2),
            pl.BlockSpec((T, bb), lane_tile2),
            pl.BlockSpec((H, S + 3), full2),
            pl.BlockSpec((H, S + 3), full2),
            pl.BlockSpec((H2, 1), full2),
            pl.BlockSpec((A, H), full2),
            pl.BlockSpec((1, H), full2),
            pl.BlockSpec((1, 1), full2),
        ],
        out_specs=[
            pl.BlockSpec((T, A, bb), lambda i: (0, 0, i)),
            pl.BlockSpec((1, bb), lane_tile2),
        ],
        out_shape=[
            jax.ShapeDtypeStruct((T, A, B), f32),
            jax.ShapeDtypeStruct((1, B), f32),
        ],
        scratch_shapes=[
            pltpu.VMEM((T * H2, S * T), f32),
            pltpu.VMEM((T * H2, 4 * T), f32),
        ],
        compiler_params=pltpu.CompilerParams(
            dimension_semantics=("arbitrary",)),
    )(SRt, HDst, actst, valst, w1at, w1ct, b1col, w2at, w2ct, b2ct)

    logp = jnp.transpose(out_tab, (2, 0, 1))
    value = val.reshape(B)
    return logp, value


# speedup vs baseline: 6.7635x; 1.0979x over previous
import jax
import jax.numpy as jnp
from jax.experimental import pallas as pl
from jax.experimental.pallas import tpu as pltpu


def _make_body(T, S, A, H, H2):
    f32 = jnp.float32

    def body(sr_ref, hds_ref, acts_ref, vals_ref,
             w1at_ref, w1ct_ref, b1a_ref, b1c_ref, w2a_ref, w2ct_ref,
             b2c_ref, out_ref, val_ref, w4s_ref, wfs_ref):
        bb = sr_ref.shape[-1]

        @pl.when(pl.program_id(0) == 0)
        def _build_interleaved_weights():
            w1fT = jnp.concatenate(
                [w1at_ref[...], w1ct_ref[...]], axis=0)
            wmain = w1fT[:, :S]
            b1row = jnp.concatenate(
                [b1a_ref[...], b1c_ref[...]], axis=1)
            ei = jax.lax.broadcasted_iota(jnp.int32, (H2, H2), 0)
            ej = jax.lax.broadcasted_iota(jnp.int32, (H2, H2), 1)
            eye2h = (ei == ej).astype(f32)
            b1col = jax.lax.dot_general(
                eye2h, b1row, (((1,), (1,)), ((), ())),
                preferred_element_type=f32)
            wsmall = jnp.concatenate(
                [w1fT[:, S:S + 3], b1col], axis=1)
            for t in range(T):
                ci = jax.lax.broadcasted_iota(jnp.int32, (S, S * T), 1)
                ri = jax.lax.broadcasted_iota(jnp.int32, (S, S * T), 0)
                sel = (ci == T * ri + t).astype(f32)
                w4s_ref[H2 * t:H2 * (t + 1), :] = jnp.dot(
                    wmain, sel, preferred_element_type=f32)
                cif = jax.lax.broadcasted_iota(jnp.int32, (4, 4 * T), 1)
                rif = jax.lax.broadcasted_iota(jnp.int32, (4, 4 * T), 0)
                self_f = (cif == T * rif + t).astype(f32)
                wfs_ref[H2 * t:H2 * (t + 1), :] = jnp.dot(
                    wsmall, self_f, preferred_element_type=f32)

        x2d = sr_ref[...].reshape(S * T, bb)
        hp4 = jnp.dot(w4s_ref[...], x2d, preferred_element_type=f32)

        feats = jnp.concatenate(
            [hds_ref[...], acts_ref[...], vals_ref[...],
             jnp.ones((T, bb), f32)], axis=0)
        hp4 = hp4 + jnp.dot(wfs_ref[...], feats, preferred_element_type=f32)

        w2a = w2a_ref[...]

        outs = []
        for t in range(T):
            ha = jnp.tanh(hp4[H2 * t:H2 * t + H, :])
            outs.append(jax.lax.dot_general(
                w2a, ha, (((0,), (0,)), ((), ())),
                preferred_element_type=f32))

        hc = jnp.tanh(hp4[H:H2, :])
        val_ref[...] = (jnp.dot(w2ct_ref[...], hc, preferred_element_type=f32)
                        + b2c_ref[...])

        m = outs[0]
        for t in range(1, T):
            m = jnp.maximum(m, outs[t])
        se = jnp.exp(outs[0] - m)
        for t in range(1, T):
            se = se + jnp.exp(outs[t] - m)
        lse = m + jnp.log(se)
        for t in range(T):
            out_ref[t, :, :] = outs[t] - lse

    return body


def _pick_b_block(B):
    for cand in (4096, 2048, 1024, 512, 256, 128):
        if B % cand == 0 and (B // cand) >= 2:
            return cand
    return B


def kernel(w1a, b1a, w2a, b2a, w1c, b1c, w2c, b2c, SR, HDs, acts, values):
    f32 = jnp.float32
    B, T, S = SR.shape
    H = w1a.shape[1]
    H2 = 2 * H
    A = w2a.shape[1]

    w1at = w1a.astype(f32).T
    w1ct = w1c.astype(f32).T
    w2ct = w2c.astype(f32).T
    b2ct = b2c.astype(f32)

    SRt = jnp.transpose(SR.astype(f32), (2, 1, 0))
    HDst = HDs.astype(f32).T
    actst = acts.astype(f32).T
    valst = values.astype(f32).T

    bb = _pick_b_block(B)
    nb = B // bb
    lane_tile2 = lambda i: (0, i)
    full2 = lambda i: (0, 0)

    out_tab, val = pl.pallas_call(
        _make_body(T, S, A, H, H2),
        grid=(nb,),
        in_specs=[
            pl.BlockSpec((S, T, bb), lambda i: (0, 0, i)),
            pl.BlockSpec((T, bb), lane_tile2),
            pl.BlockSpec((T, bb), lane_tile2),
            pl.BlockSpec((T, bb), lane_tile2),
            pl.BlockSpec((H, S + 3), full2),
            pl.BlockSpec((H, S + 3), full2),
            pl.BlockSpec((1, H), full2),
            pl.BlockSpec((1, H), full2),
            pl.BlockSpec((H, A), full2),
            pl.BlockSpec((1, H), full2),
            pl.BlockSpec((1, 1), full2),
        ],
        out_specs=[
            pl.BlockSpec((T, A, bb), lambda i: (0, 0, i)),
            pl.BlockSpec((1, bb), lane_tile2),
        ],
        out_shape=[
            jax.ShapeDtypeStruct((T, A, B), f32),
            jax.ShapeDtypeStruct((1, B), f32),
        ],
        scratch_shapes=[
            pltpu.VMEM((T * H2, S * T), f32),
            pltpu.VMEM((T * H2, 4 * T), f32),
        ],
        compiler_params=pltpu.CompilerParams(
            dimension_semantics=("arbitrary",)),
    )(SRt, HDst, actst, valst, w1at, w1ct, b1a.astype(f32), b1c.astype(f32),
      w2a.astype(f32), w2ct, b2ct)

    logp = jnp.transpose(out_tab, (2, 0, 1))
    value = val.reshape(B)
    return logp, value
